# baseline (speedup 1.0000x reference)
import functools

import jax
import jax.numpy as jnp
from jax import lax
from jax.experimental import pallas as pl
from jax.experimental.pallas import tpu as pltpu
from jax.experimental.pallas import tpu_sc as plsc

LATENT = 32
ROWS = 1000000
BATCH = 16384
NC, NS, L = 2, 16, 16
NW = NC * NS
B_PER_W = BATCH // NW
BLK = 128
QN = B_PER_W // BLK
G_PER_BLK = BLK // L


def _mf_body(uid_hbm, iid_hbm, ut_hbm, it_hbm, out_hbm,
             uid_v, iid_v, uidx, iidx, uval, ival, out_v, sem):
    wid = lax.axis_index("s") * NC + lax.axis_index("c")
    base = wid * B_PER_W

    pltpu.sync_copy(uid_hbm.at[pl.ds(base, B_PER_W)], uid_v)
    pltpu.sync_copy(iid_hbm.at[pl.ds(base, B_PER_W)], iid_v)

    @pl.loop(0, QN)
    def _build(q):
        for g in range(G_PER_BLK):
            uvec = uid_v[pl.ds(q * BLK + g * L, L)]
            ivec = iid_v[pl.ds(q * BLK + g * L, L)]
            for d in range(LATENT):
                off = jnp.full((L,), d * ROWS, jnp.int32)
                uidx[d * QN + q, pl.ds(g * L, L)] = uvec + off
                iidx[d * QN + q, pl.ds(g * L, L)] = ivec + off

    descs = []
    for r in range(LATENT * QN):
        descs.append(pltpu.async_copy(
            ut_hbm.at[uidx.at[r]], uval.at[r], sem))
        descs.append(pltpu.async_copy(
            it_hbm.at[iidx.at[r]], ival.at[r], sem))
    for dsc in descs:
        dsc.wait()

    @pl.loop(0, QN)
    def _compute(q):
        for g in range(G_PER_BLK):
            acc = jnp.zeros((L,), jnp.float32)
            for d in range(LATENT):
                u = uval[d * QN + q, pl.ds(g * L, L)]
                v = ival[d * QN + q, pl.ds(g * L, L)]
                acc = acc + u * v
            out_v[pl.ds(q * BLK + g * L, L)] = acc

    pltpu.sync_copy(out_v, out_hbm.at[pl.ds(base, B_PER_W)])


@jax.jit
def _mf(user_id, item_id, ut1, it1):
    mesh = plsc.VectorSubcoreMesh(
        core_axis_name="c", subcore_axis_name="s",
        num_cores=NC, num_subcores=NS)
    run = functools.partial(
        pl.kernel,
        out_type=jax.ShapeDtypeStruct((BATCH,), jnp.float32),
        mesh=mesh,
        compiler_params=pltpu.CompilerParams(needs_layout_passes=False),
        scratch_types=[
            pltpu.VMEM((B_PER_W,), jnp.int32),
            pltpu.VMEM((B_PER_W,), jnp.int32),
            pltpu.VMEM((LATENT * QN, BLK), jnp.int32),
            pltpu.VMEM((LATENT * QN, BLK), jnp.int32),
            pltpu.VMEM((LATENT * QN, BLK), jnp.float32),
            pltpu.VMEM((LATENT * QN, BLK), jnp.float32),
            pltpu.VMEM((B_PER_W,), jnp.float32),
            pltpu.SemaphoreType.DMA,
        ],
    )(_mf_body)
    return run(user_id, item_id, ut1, it1)


def kernel(user_id, item_id, user_table, item_table):
    ut1 = user_table.T.reshape(-1)
    it1 = item_table.T.reshape(-1)
    return _mf(user_id.astype(jnp.int32), item_id.astype(jnp.int32), ut1, it1)


# speedup vs baseline: 20.1045x; 20.1045x over previous
import functools

import jax
import jax.numpy as jnp
from jax import lax
from jax.experimental import pallas as pl
from jax.experimental.pallas import tpu as pltpu
from jax.experimental.pallas import tpu_sc as plsc

LATENT = 32
ROWS = 1000000
ROWS_D = 999936
TAIL_N = ROWS - ROWS_D
BATCH = 16384
NC, NS, L = 2, 16, 16
NW = NC * NS
B_PER_W = BATCH // NW
BLK = 128
QN = B_PER_W // BLK
G_PER_BLK = BLK // L

CW = 32256
DT_N = ROWS_D // CW


def _detile_body(in_ref, out_hbm, sem):
    k = pl.program_id(0)
    cps = []
    for d in range(LATENT):
        cps.append(pltpu.make_async_copy(
            in_ref.at[d],
            out_hbm.at[pl.ds(d * ROWS_D + k * CW, CW)],
            sem))
    for c in cps:
        c.start()
    for c in cps:
        c.wait()


def _detile(tT):
    return pl.pallas_call(
        _detile_body,
        grid=(DT_N,),
        in_specs=[pl.BlockSpec((LATENT, CW), lambda k: (0, k))],
        out_specs=pl.BlockSpec(memory_space=pl.ANY),
        out_shape=jax.ShapeDtypeStruct((LATENT * ROWS_D,), jnp.float32),
        scratch_shapes=[pltpu.SemaphoreType.DMA],
    )(tT)


def _mf_body(uid_hbm, iid_hbm, ut_hbm, it_hbm, utail_hbm, itail_hbm, out_hbm,
             uid_v, iid_v, uidx, iidx, uval, ival, utail_v, itail_v,
             out_v, sem):
    wid = lax.axis_index("s") * NC + lax.axis_index("c")
    base = wid * B_PER_W

    pltpu.sync_copy(uid_hbm.at[pl.ds(base, B_PER_W)], uid_v)
    pltpu.sync_copy(iid_hbm.at[pl.ds(base, B_PER_W)], iid_v)
    pltpu.sync_copy(utail_hbm, utail_v)
    pltpu.sync_copy(itail_hbm, itail_v)

    rmax = jnp.full((L,), ROWS_D - 1, jnp.int32)

    @pl.loop(0, QN)
    def _build(q):
        for g in range(G_PER_BLK):
            uvec = jnp.minimum(uid_v[pl.ds(q * BLK + g * L, L)], rmax)
            ivec = jnp.minimum(iid_v[pl.ds(q * BLK + g * L, L)], rmax)
            for d in range(LATENT):
                off = jnp.full((L,), d * ROWS_D, jnp.int32)
                uidx[d * QN + q, pl.ds(g * L, L)] = uvec + off
                iidx[d * QN + q, pl.ds(g * L, L)] = ivec + off

    descs = []
    for r in range(LATENT * QN):
        descs.append(pltpu.async_copy(
            ut_hbm.at[uidx.at[r]], uval.at[r], sem))
        descs.append(pltpu.async_copy(
            it_hbm.at[iidx.at[r]], ival.at[r], sem))
    for dsc in descs:
        dsc.wait()

    tlo = jnp.full((L,), ROWS_D, jnp.int32)

    @pl.loop(0, QN)
    def _compute(q):
        for g in range(G_PER_BLK):
            uvec = uid_v[pl.ds(q * BLK + g * L, L)]
            ivec = iid_v[pl.ds(q * BLK + g * L, L)]
            umask = uvec >= tlo
            imask = ivec >= tlo
            uloc = jnp.where(umask, (uvec - tlo) * LATENT, 0)
            iloc = jnp.where(imask, (ivec - tlo) * LATENT, 0)
            acc = jnp.zeros((L,), jnp.float32)
            for d in range(LATENT):
                u = uval[d * QN + q, pl.ds(g * L, L)]
                v = ival[d * QN + q, pl.ds(g * L, L)]
                dv = jnp.full((L,), d, jnp.int32)
                ut = plsc.load_gather(utail_v, [uloc + dv])
                vt = plsc.load_gather(itail_v, [iloc + dv])
                u = jnp.where(umask, ut, u)
                v = jnp.where(imask, vt, v)
                acc = acc + u * v
            out_v[pl.ds(q * BLK + g * L, L)] = acc

    pltpu.sync_copy(out_v, out_hbm.at[pl.ds(base, B_PER_W)])


@jax.jit
def _mf(user_id, item_id, utT, itT, utail, itail):
    ut1 = _detile(utT)
    it1 = _detile(itT)
    mesh = plsc.VectorSubcoreMesh(
        core_axis_name="c", subcore_axis_name="s",
        num_cores=NC, num_subcores=NS)
    run = functools.partial(
        pl.kernel,
        out_type=jax.ShapeDtypeStruct((BATCH,), jnp.float32),
        mesh=mesh,
        compiler_params=pltpu.CompilerParams(needs_layout_passes=False),
        scratch_types=[
            pltpu.VMEM((B_PER_W,), jnp.int32),
            pltpu.VMEM((B_PER_W,), jnp.int32),
            pltpu.VMEM((LATENT * QN, BLK), jnp.int32),
            pltpu.VMEM((LATENT * QN, BLK), jnp.int32),
            pltpu.VMEM((LATENT * QN, BLK), jnp.float32),
            pltpu.VMEM((LATENT * QN, BLK), jnp.float32),
            pltpu.VMEM((TAIL_N * LATENT,), jnp.float32),
            pltpu.VMEM((TAIL_N * LATENT,), jnp.float32),
            pltpu.VMEM((B_PER_W,), jnp.float32),
            pltpu.SemaphoreType.DMA,
        ],
    )(_mf_body)
    return run(user_id, item_id, ut1, it1, utail, itail)


def kernel(user_id, item_id, user_table, item_table):
    utail = user_table[ROWS_D:].reshape(-1)
    itail = item_table[ROWS_D:].reshape(-1)
    return _mf(user_id.astype(jnp.int32), item_id.astype(jnp.int32),
               user_table.T, item_table.T, utail, itail)
